# baseline (speedup 1.0000x reference)
import functools

import jax
import jax.numpy as jnp
from jax import lax
from jax.experimental import pallas as pl
from jax.experimental.pallas import tpu as pltpu
from jax.experimental.pallas import tpu_sc as plsc

_N = 10000
_H = 128
_K = 8
_C = 40
_NC = 2
_NS = 16
_B = 64
_NBUF = 4
_NSTG = 4
_SCH = 40
_N0 = _NSTG * _SCH
_CH_ARR = _NS * _N0
_ROWS_SP = 10240
_RPT = _ROWS_SP // _NS


def _sc_spmm(h, src_r, dst_r):
    mesh = plsc.VectorSubcoreMesh(
        core_axis_name="c", subcore_axis_name="s", num_cores=1
    )

    @functools.partial(
        pl.kernel,
        mesh=mesh,
        out_type=jax.ShapeDtypeStruct((_NS, _RPT, _H), jnp.float32),
        scratch_types=[
            pltpu.VMEM((_SCH, _B), jnp.int32),
            pltpu.VMEM((_SCH, _B), jnp.int32),
            pltpu.VMEM((_NBUF, _B, _H), jnp.float32),
            pltpu.VMEM_SHARED((_ROWS_SP, _H), jnp.float32),
            pltpu.SemaphoreType.DMA((_NBUF,)),
            pltpu.SemaphoreType.DMA((_NBUF,)),
        ],
    )
    def k(h_hbm, src_hbm, dst_hbm, out_hbm, src_v, dst_v, rows, acc_sp,
          gsem, ssem):
        s = lax.axis_index("s")

        def zrow(r, carry):
            for cc in range(_H // 16):
                rows[0, r, pl.ds(cc * 16, 16)] = jnp.zeros((16,), jnp.float32)
            return carry

        lax.fori_loop(0, _B, zrow, 0)
        for t in range(_RPT // _B):
            pltpu.sync_copy(
                rows.at[0], acc_sp.at[pl.ds(s * _RPT + t * _B, _B)]
            )
        plsc.subcore_barrier()

        for stage in range(_NSTG):
            if True:
                base = pl.multiple_of(s * _N0 + stage * _SCH, 8)
                pltpu.sync_copy(src_hbm.at[pl.ds(base, _SCH)], src_v)
                pltpu.sync_copy(dst_hbm.at[pl.ds(base, _SCH)], dst_v)
                for b in range(_NBUF):
                    pltpu.async_copy(
                        h_hbm.at[dst_v.at[b]], rows.at[b], gsem.at[b]
                    )

                def body(i, carry):
                    j0 = i * _NBUF
                    for b in range(_NBUF):
                        pltpu.make_async_copy(
                            h_hbm.at[dst_v.at[j0 + b]], rows.at[b],
                            gsem.at[b],
                        ).wait()
                        pltpu.async_copy(
                            rows.at[b], acc_sp.at[src_v.at[j0 + b]],
                            ssem.at[b], add=True,
                        )
                    for b in range(_NBUF):
                        pltpu.make_async_copy(
                            rows.at[b], acc_sp.at[src_v.at[j0 + b]],
                            ssem.at[b],
                        ).wait()

                        @pl.when(j0 + _NBUF + b < _SCH)
                        def _start_next():
                            pltpu.async_copy(
                                h_hbm.at[dst_v.at[j0 + _NBUF + b]],
                                rows.at[b],
                                gsem.at[b],
                            )

                    return carry

                lax.fori_loop(0, _SCH // _NBUF, body, 0)

        plsc.subcore_barrier()

        pltpu.sync_copy(acc_sp.at[pl.ds(s * _RPT, _RPT)], out_hbm.at[s])

    return k(h, src_r, dst_r)


def _pre_body(x_ref, w_ref, b_ref, o_ref):
    y = jnp.dot(x_ref[...], w_ref[...], preferred_element_type=jnp.float32)
    o_ref[...] = jnp.maximum(y + b_ref[...], 0.0)


def _tc_pre(x, w, b):
    rows = 2000
    return pl.pallas_call(
        _pre_body,
        grid=(_N // rows,),
        in_specs=[
            pl.BlockSpec((rows, _H), lambda i: (i, 0)),
            pl.BlockSpec((_H, _H), lambda i: (0, 0)),
            pl.BlockSpec((1, _H), lambda i: (0, 0)),
        ],
        out_specs=pl.BlockSpec((rows, _H), lambda i: (i, 0)),
        out_shape=jax.ShapeDtypeStruct((_N, _H), jnp.float32),
    )(x, w, b.reshape(1, _H))


def _combine_body(final, h_ref, hi_ref, wc_ref, bc_ref, wk_ref,
                  w2_ref, b2_ref, o_ref):
    h = h_ref[...]
    hi = hi_ref[...]
    logit = jnp.dot(h, wc_ref[...], preferred_element_type=jnp.float32)
    logit = logit + bc_ref[...]
    m = jnp.max(logit, axis=-1, keepdims=True)
    e = jnp.exp(logit - m)
    z = e / jnp.sum(e, axis=-1, keepdims=True)
    acc = h
    for k in range(_K):
        t = jnp.dot(hi, wk_ref[k], preferred_element_type=jnp.float32)
        acc = acc + z[:, k : k + 1] * t
    hn = jnp.maximum(acc, 0.0)
    if final:
        y = jnp.dot(hn, w2_ref[...], preferred_element_type=jnp.float32)
        o_ref[...] = y + b2_ref[...]
    else:
        o_ref[...] = hn


def _tc_combine(final, h, parts, wcp, bcp, wk, w2p, b2p):
    rows = 2000
    grid = (_N // rows,)
    return pl.pallas_call(
        functools.partial(_combine_body, final),
        grid=grid,
        in_specs=[
            pl.BlockSpec((rows, _H), lambda i: (i, 0)),
            pl.BlockSpec((rows, _H), lambda i: (i, 0)),
            pl.BlockSpec((_H, _H), lambda i: (0, 0)),
            pl.BlockSpec((1, _H), lambda i: (0, 0)),
            pl.BlockSpec((_K, _H, _H), lambda i: (0, 0, 0)),
            pl.BlockSpec((_H, _H), lambda i: (0, 0)),
            pl.BlockSpec((1, _H), lambda i: (0, 0)),
        ],
        out_specs=pl.BlockSpec((rows, _H), lambda i: (i, 0)),
        out_shape=jax.ShapeDtypeStruct((_N, _H), jnp.float32),
    )(h, parts, wcp, bcp.reshape(1, _H), wk, w2p, b2p.reshape(1, _H))


def kernel(x, edge_index, W1, b1, Wc, bc, Wk, W2, b2):
    e = edge_index.shape[1]
    pad = _CH_ARR * _B - e
    src = jnp.concatenate([edge_index[0], jnp.full((pad,), _N, jnp.int32)])
    dst = jnp.concatenate([edge_index[1], jnp.zeros((pad,), jnp.int32)])
    src_r = src.reshape(_CH_ARR, _B)
    dst_r = dst.reshape(_CH_ARR, _B)

    wcp = jnp.zeros((2, _H, _H), jnp.float32).at[:, :, : _K].set(Wc)
    bcp = jnp.full((2, _H), -1e30, jnp.float32).at[:, : _K].set(bc)
    w2p = jnp.zeros((_H, _H), jnp.float32).at[:, : _C].set(W2)
    b2p = jnp.zeros((_H,), jnp.float32).at[: _C].set(b2)

    h = _tc_pre(x, W1, b1)
    for i in range(2):
        parts = _sc_spmm(h, src_r, dst_r).reshape(_ROWS_SP, _H)
        h = _tc_combine(i == 1, h, parts, wcp[i], bcp[i], Wk[i], w2p, b2p)
    return h[:, : _C]


# speedup vs baseline: 1.0164x; 1.0164x over previous
import functools

import jax
import jax.numpy as jnp
from jax import lax
from jax.experimental import pallas as pl
from jax.experimental.pallas import tpu as pltpu
from jax.experimental.pallas import tpu_sc as plsc

_N = 10000
_H = 128
_K = 8
_C = 40
_NC = 2
_NS = 16
_B = 128
_NBUF = 2
_NSTG = 5
_SCH = 16
_N0 = _NSTG * _SCH
_CH_ARR = _NS * _N0
_ROWS_SP = 10240
_RPT = _ROWS_SP // _NS


def _sc_spmm(h, src_r, dst_r):
    mesh = plsc.VectorSubcoreMesh(
        core_axis_name="c", subcore_axis_name="s", num_cores=1
    )

    @functools.partial(
        pl.kernel,
        mesh=mesh,
        out_type=jax.ShapeDtypeStruct((_NS, _RPT, _H), jnp.float32),
        scratch_types=[
            pltpu.VMEM((_SCH, _B), jnp.int32),
            pltpu.VMEM((_SCH, _B), jnp.int32),
            pltpu.VMEM((_NBUF, _B, _H), jnp.float32),
            pltpu.VMEM_SHARED((_ROWS_SP, _H), jnp.float32),
            pltpu.SemaphoreType.DMA((_NBUF,)),
            pltpu.SemaphoreType.DMA((_NBUF,)),
        ],
    )
    def k(h_hbm, src_hbm, dst_hbm, out_hbm, src_v, dst_v, rows, acc_sp,
          gsem, ssem):
        s = lax.axis_index("s")

        def zrow(r, carry):
            for cc in range(_H // 16):
                rows[0, r, pl.ds(cc * 16, 16)] = jnp.zeros((16,), jnp.float32)
            return carry

        lax.fori_loop(0, _B, zrow, 0)
        for t in range(_RPT // _B):
            pltpu.sync_copy(
                rows.at[0], acc_sp.at[pl.ds(s * _RPT + t * _B, _B)]
            )
        plsc.subcore_barrier()

        for stage in range(_NSTG):
            if True:
                base = pl.multiple_of(s * _N0 + stage * _SCH, 8)
                pltpu.sync_copy(src_hbm.at[pl.ds(base, _SCH)], src_v)
                pltpu.sync_copy(dst_hbm.at[pl.ds(base, _SCH)], dst_v)
                for b in range(_NBUF):
                    pltpu.async_copy(
                        h_hbm.at[dst_v.at[b]], rows.at[b], gsem.at[b]
                    )

                def body(i, carry):
                    j0 = i * _NBUF
                    for b in range(_NBUF):
                        pltpu.make_async_copy(
                            h_hbm.at[dst_v.at[j0 + b]], rows.at[b],
                            gsem.at[b],
                        ).wait()
                        pltpu.async_copy(
                            rows.at[b], acc_sp.at[src_v.at[j0 + b]],
                            ssem.at[b], add=True,
                        )
                    for b in range(_NBUF):
                        pltpu.make_async_copy(
                            rows.at[b], acc_sp.at[src_v.at[j0 + b]],
                            ssem.at[b],
                        ).wait()

                        @pl.when(j0 + _NBUF + b < _SCH)
                        def _start_next():
                            pltpu.async_copy(
                                h_hbm.at[dst_v.at[j0 + _NBUF + b]],
                                rows.at[b],
                                gsem.at[b],
                            )

                    return carry

                lax.fori_loop(0, _SCH // _NBUF, body, 0)

        plsc.subcore_barrier()

        pltpu.sync_copy(acc_sp.at[pl.ds(s * _RPT, _RPT)], out_hbm.at[s])

    return k(h, src_r, dst_r)


def _pre_body(x_ref, w_ref, b_ref, o_ref):
    y = jnp.dot(x_ref[...], w_ref[...], preferred_element_type=jnp.float32)
    o_ref[...] = jnp.maximum(y + b_ref[...], 0.0)


def _tc_pre(x, w, b):
    rows = 2000
    return pl.pallas_call(
        _pre_body,
        grid=(_N // rows,),
        in_specs=[
            pl.BlockSpec((rows, _H), lambda i: (i, 0)),
            pl.BlockSpec((_H, _H), lambda i: (0, 0)),
            pl.BlockSpec((1, _H), lambda i: (0, 0)),
        ],
        out_specs=pl.BlockSpec((rows, _H), lambda i: (i, 0)),
        out_shape=jax.ShapeDtypeStruct((_N, _H), jnp.float32),
    )(x, w, b.reshape(1, _H))


def _combine_body(final, h_ref, hi_ref, wc_ref, bc_ref, wk_ref,
                  w2_ref, b2_ref, o_ref):
    h = h_ref[...]
    hi = hi_ref[...]
    logit = jnp.dot(h, wc_ref[...], preferred_element_type=jnp.float32)
    logit = logit + bc_ref[...]
    m = jnp.max(logit, axis=-1, keepdims=True)
    e = jnp.exp(logit - m)
    z = e / jnp.sum(e, axis=-1, keepdims=True)
    acc = h
    for k in range(_K):
        t = jnp.dot(hi, wk_ref[k], preferred_element_type=jnp.float32)
        acc = acc + z[:, k : k + 1] * t
    hn = jnp.maximum(acc, 0.0)
    if final:
        y = jnp.dot(hn, w2_ref[...], preferred_element_type=jnp.float32)
        o_ref[...] = y + b2_ref[...]
    else:
        o_ref[...] = hn


def _tc_combine(final, h, parts, wcp, bcp, wk, w2p, b2p):
    rows = 2000
    grid = (_N // rows,)
    return pl.pallas_call(
        functools.partial(_combine_body, final),
        grid=grid,
        in_specs=[
            pl.BlockSpec((rows, _H), lambda i: (i, 0)),
            pl.BlockSpec((rows, _H), lambda i: (i, 0)),
            pl.BlockSpec((_H, _H), lambda i: (0, 0)),
            pl.BlockSpec((1, _H), lambda i: (0, 0)),
            pl.BlockSpec((_K, _H, _H), lambda i: (0, 0, 0)),
            pl.BlockSpec((_H, _H), lambda i: (0, 0)),
            pl.BlockSpec((1, _H), lambda i: (0, 0)),
        ],
        out_specs=pl.BlockSpec((rows, _H), lambda i: (i, 0)),
        out_shape=jax.ShapeDtypeStruct((_N, _H), jnp.float32),
    )(h, parts, wcp, bcp.reshape(1, _H), wk, w2p, b2p.reshape(1, _H))


def kernel(x, edge_index, W1, b1, Wc, bc, Wk, W2, b2):
    e = edge_index.shape[1]
    pad = _CH_ARR * _B - e
    src = jnp.concatenate([edge_index[0], jnp.full((pad,), _N, jnp.int32)])
    dst = jnp.concatenate([edge_index[1], jnp.zeros((pad,), jnp.int32)])
    src_r = src.reshape(_CH_ARR, _B)
    dst_r = dst.reshape(_CH_ARR, _B)

    wcp = jnp.zeros((2, _H, _H), jnp.float32).at[:, :, : _K].set(Wc)
    bcp = jnp.full((2, _H), -1e30, jnp.float32).at[:, : _K].set(bc)
    w2p = jnp.zeros((_H, _H), jnp.float32).at[:, : _C].set(W2)
    b2p = jnp.zeros((_H,), jnp.float32).at[: _C].set(b2)

    h = _tc_pre(x, W1, b1)
    for i in range(2):
        parts = _sc_spmm(h, src_r, dst_r).reshape(_ROWS_SP, _H)
        h = _tc_combine(i == 1, h, parts, wcp[i], bcp[i], Wk[i], w2p, b2p)
    return h[:, : _C]


# speedup vs baseline: 1.3358x; 1.3143x over previous
import functools

import jax
import jax.numpy as jnp
from jax import lax
from jax.experimental import pallas as pl
from jax.experimental.pallas import tpu as pltpu
from jax.experimental.pallas import tpu_sc as plsc

_N = 10000
_H = 128
_K = 8
_C = 40
_NC = 2
_NS = 16
_B = 128
_NBUF = 2
_NSTG = 9
_SCH = 8
_N0 = _NSTG * _SCH
_N1 = _SCH
_CH_SC0 = _NS * _N0
_CH_ARR = _CH_SC0 + _NS * _N1
_ROWS_SP = 10240
_RPT = _ROWS_SP // _NS


def _sc_spmm(h, src_r, dst_r):
    mesh = plsc.VectorSubcoreMesh(core_axis_name="c", subcore_axis_name="s")

    @functools.partial(
        pl.kernel,
        mesh=mesh,
        out_type=jax.ShapeDtypeStruct((_NC, _NS, _RPT, _H), jnp.float32),
        scratch_types=[
            pltpu.VMEM((_SCH, _B), jnp.int32),
            pltpu.VMEM((_SCH, _B), jnp.int32),
            pltpu.VMEM((_NBUF, _B, _H), jnp.float32),
            pltpu.VMEM_SHARED((_ROWS_SP, _H), jnp.float32),
            pltpu.SemaphoreType.DMA((_NBUF,)),
            pltpu.SemaphoreType.DMA((_NBUF,)),
        ],
    )
    def k(h_hbm, src_hbm, dst_hbm, out_hbm, src_v, dst_v, rows, acc_sp,
          gsem, ssem):
        c = lax.axis_index("c")
        s = lax.axis_index("s")

        def zrow(r, carry):
            for cc in range(_H // 16):
                rows[0, r, pl.ds(cc * 16, 16)] = jnp.zeros((16,), jnp.float32)
            return carry

        lax.fori_loop(0, _B, zrow, 0)
        for t in range(_RPT // _B):
            pltpu.sync_copy(
                rows.at[0], acc_sp.at[pl.ds(s * _RPT + t * _B, _B)]
            )
        plsc.subcore_barrier()

        nstg = jnp.where(c == 0, _NSTG, 1)
        tbase = jnp.where(c == 0, s * _N0, _CH_SC0 + s * _N1)
        for stage in range(_NSTG):

            @pl.when(stage < nstg)
            def _stage():
                base = pl.multiple_of(tbase + stage * _SCH, 8)
                pltpu.sync_copy(src_hbm.at[pl.ds(base, _SCH)], src_v)
                pltpu.sync_copy(dst_hbm.at[pl.ds(base, _SCH)], dst_v)
                for b in range(_NBUF):
                    pltpu.async_copy(
                        h_hbm.at[dst_v.at[b]], rows.at[b], gsem.at[b]
                    )

                def body(i, carry):
                    j0 = i * _NBUF
                    for b in range(_NBUF):
                        pltpu.make_async_copy(
                            h_hbm.at[dst_v.at[j0 + b]], rows.at[b],
                            gsem.at[b],
                        ).wait()
                        pltpu.async_copy(
                            rows.at[b], acc_sp.at[src_v.at[j0 + b]],
                            ssem.at[b], add=True,
                        )
                    for b in range(_NBUF):
                        pltpu.make_async_copy(
                            rows.at[b], acc_sp.at[src_v.at[j0 + b]],
                            ssem.at[b],
                        ).wait()

                        @pl.when(j0 + _NBUF + b < _SCH)
                        def _start_next():
                            pltpu.async_copy(
                                h_hbm.at[dst_v.at[j0 + _NBUF + b]],
                                rows.at[b],
                                gsem.at[b],
                            )

                    return carry

                lax.fori_loop(0, _SCH // _NBUF, body, 0)

        plsc.subcore_barrier()

        pltpu.sync_copy(
            acc_sp.at[pl.ds(s * _RPT, _RPT)], out_hbm.at[c, s]
        )

    return k(h, src_r, dst_r)


def _pre_body(x_ref, w_ref, b_ref, o_ref):
    y = jnp.dot(x_ref[...], w_ref[...], preferred_element_type=jnp.float32)
    o_ref[...] = jnp.maximum(y + b_ref[...], 0.0)


def _tc_pre(x, w, b):
    rows = 2000
    return pl.pallas_call(
        _pre_body,
        grid=(_N // rows,),
        in_specs=[
            pl.BlockSpec((rows, _H), lambda i: (i, 0)),
            pl.BlockSpec((_H, _H), lambda i: (0, 0)),
            pl.BlockSpec((1, _H), lambda i: (0, 0)),
        ],
        out_specs=pl.BlockSpec((rows, _H), lambda i: (i, 0)),
        out_shape=jax.ShapeDtypeStruct((_N, _H), jnp.float32),
    )(x, w, b.reshape(1, _H))


def _combine_body(final, h_ref, hi0_ref, hi1_ref, wc_ref, bc_ref, wk_ref,
                  w2_ref, b2_ref, o_ref):
    h = h_ref[...]
    hi = hi0_ref[0] + hi1_ref[0]
    logit = jnp.dot(h, wc_ref[...], preferred_element_type=jnp.float32)
    logit = logit + bc_ref[...]
    m = jnp.max(logit, axis=-1, keepdims=True)
    e = jnp.exp(logit - m)
    z = e / jnp.sum(e, axis=-1, keepdims=True)
    acc = h
    for k in range(_K):
        t = jnp.dot(hi, wk_ref[k], preferred_element_type=jnp.float32)
        acc = acc + z[:, k : k + 1] * t
    hn = jnp.maximum(acc, 0.0)
    if final:
        y = jnp.dot(hn, w2_ref[...], preferred_element_type=jnp.float32)
        o_ref[...] = y + b2_ref[...]
    else:
        o_ref[...] = hn


def _tc_combine(final, h, parts, wcp, bcp, wk, w2p, b2p):
    rows = 2000
    grid = (_N // rows,)
    return pl.pallas_call(
        functools.partial(_combine_body, final),
        grid=grid,
        in_specs=[
            pl.BlockSpec((rows, _H), lambda i: (i, 0)),
            pl.BlockSpec((1, rows, _H), lambda i: (0, i, 0)),
            pl.BlockSpec((1, rows, _H), lambda i: (1, i, 0)),
            pl.BlockSpec((_H, _H), lambda i: (0, 0)),
            pl.BlockSpec((1, _H), lambda i: (0, 0)),
            pl.BlockSpec((_K, _H, _H), lambda i: (0, 0, 0)),
            pl.BlockSpec((_H, _H), lambda i: (0, 0)),
            pl.BlockSpec((1, _H), lambda i: (0, 0)),
        ],
        out_specs=pl.BlockSpec((rows, _H), lambda i: (i, 0)),
        out_shape=jax.ShapeDtypeStruct((_N, _H), jnp.float32),
    )(h, parts, parts, wcp, bcp.reshape(1, _H), wk, w2p, b2p.reshape(1, _H))


def kernel(x, edge_index, W1, b1, Wc, bc, Wk, W2, b2):
    e = edge_index.shape[1]
    pad = _CH_ARR * _B - e
    src = jnp.concatenate([edge_index[0], jnp.full((pad,), _N, jnp.int32)])
    dst = jnp.concatenate([edge_index[1], jnp.zeros((pad,), jnp.int32)])
    src_r = src.reshape(_CH_ARR, _B)
    dst_r = dst.reshape(_CH_ARR, _B)

    wcp = jnp.zeros((2, _H, _H), jnp.float32).at[:, :, : _K].set(Wc)
    bcp = jnp.full((2, _H), -1e30, jnp.float32).at[:, : _K].set(bc)
    w2p = jnp.zeros((_H, _H), jnp.float32).at[:, : _C].set(W2)
    b2p = jnp.zeros((_H,), jnp.float32).at[: _C].set(b2)

    h = _tc_pre(x, W1, b1)
    for i in range(2):
        parts = _sc_spmm(h, src_r, dst_r).reshape(_NC, _ROWS_SP, _H)
        h = _tc_combine(i == 1, h, parts, wcp[i], bcp[i], Wk[i], w2p, b2p)
    return h[:, : _C]
